# Initial kernel scaffold; baseline (speedup 1.0000x reference)
import jax
import jax.numpy as jnp
from jax.experimental import pallas as pl


def kernel(node_attr, edge_index, edge_attr, edge_sh, Wq, W1k, b1k, W2k, Wk, W1v, b1v, W2v, Wv, Wdot):
    raise NotImplementedError("write your pallas kernel here")


# speedup vs baseline: 3.3668x; 3.3668x over previous
import functools

import jax
import jax.numpy as jnp
from jax import lax
from jax.experimental import pallas as pl
from jax.experimental.pallas import tpu as pltpu
from jax.experimental.pallas import tpu_sc as plsc

_NC = 2
_NS = 16
_NW = _NC * _NS
_CH = 128
_BE = 1024
_D_IN = 128
_D_SH = 4
_D_Q = 80
_D_OUT = 128


def _prep_body(na_ref, wq_ref, wk_ref, wdot_ref, q_ref, wkd_ref):
    q_ref[...] = jnp.dot(na_ref[...], wq_ref[...],
                         preferred_element_type=jnp.float32)
    wkd_ref[...] = lax.dot_general(
        wk_ref[...], wdot_ref[...], (((1,), (1,)), ((), ())),
        preferred_element_type=jnp.float32)


def _dense_body(e_real, feat_ref, qd_ref, ea_ref, sh_ref, dstm_ref, w1_ref,
                b1_ref, w2k_ref, w2v_ref, wkd_ref, wv_ref, o1_ref, o2_ref):
    b = feat_ref.shape[0]
    h = jnp.maximum(
        jnp.dot(ea_ref[...], w1_ref[...], preferred_element_type=jnp.float32)
        + b1_ref[...], 0.0)
    wk = jnp.dot(h[:, :64], w2k_ref[...], preferred_element_type=jnp.float32)
    wv = jnp.dot(h[:, 64:], w2v_ref[...], preferred_element_type=jnp.float32)
    f = feat_ref[...]
    sh = sh_ref[...]
    tp = jnp.concatenate([f * sh[:, s:s + 1] for s in range(_D_SH)], axis=1)
    kd = jnp.dot(tp * wk, wkd_ref[...], preferred_element_type=jnp.float32)
    v = jnp.dot(tp * wv, wv_ref[...], preferred_element_type=jnp.float32)
    alpha = jnp.sum(qd_ref[...] * kd, axis=1, keepdims=True) * (_D_Q ** -0.5)
    rows = lax.broadcasted_iota(jnp.int32, (b, 1), 0) + pl.program_id(0) * b
    ealpha = jnp.where(rows < e_real, jnp.exp(alpha), 0.0)
    o1_ref[...] = ealpha * v
    lanes = lax.broadcasted_iota(jnp.int32, (b, _D_OUT), 1)
    o2_ref[...] = jnp.where(lanes == dstm_ref[...], ealpha, 0.0)


def _fin_body(p1_ref, p2_ref, o_ref):
    num = p1_ref[0] + p1_ref[1]
    den = p2_ref[0] + p2_ref[1]
    safe = jnp.where(den > 0.0, den, 1.0)
    o_ref[...] = jnp.where(den > 0.0, num / safe, 0.0)


def _sc_gather(node_attr, q, src2d, dst2d, e_pad):
    n_ch = e_pad // (_NW * _CH)
    mesh = plsc.VectorSubcoreMesh(core_axis_name="c", subcore_axis_name="s",
                                  num_cores=_NC, num_subcores=_NS)

    @functools.partial(
        pl.kernel,
        out_type=(jax.ShapeDtypeStruct((e_pad, _D_IN), jnp.float32),
                  jax.ShapeDtypeStruct((e_pad, _D_IN), jnp.float32)),
        mesh=mesh,
        scratch_types=(pltpu.VMEM((e_pad // (_NW * _CH), _CH), jnp.int32),
                       pltpu.VMEM((e_pad // (_NW * _CH), _CH), jnp.int32),
                       pltpu.VMEM((_CH, _D_IN), jnp.float32),
                       pltpu.VMEM((_CH, _D_IN), jnp.float32)),
    )
    def gather_k(node_hbm, q_hbm, src_hbm, dst_hbm, feat_out, qd_out,
                 src_v, dst_v, fbuf, qbuf):
        wid = lax.axis_index("s") * _NC + lax.axis_index("c")
        ch0 = wid * n_ch
        pltpu.sync_copy(src_hbm.at[pl.ds(ch0, n_ch)], src_v)
        pltpu.sync_copy(dst_hbm.at[pl.ds(ch0, n_ch)], dst_v)

        def body(j, carry):
            row = ch0 + j
            pltpu.sync_copy(node_hbm.at[src_v.at[j]], fbuf)
            pltpu.sync_copy(q_hbm.at[dst_v.at[j]], qbuf)
            pltpu.sync_copy(fbuf, feat_out.at[pl.ds(row * _CH, _CH)])
            pltpu.sync_copy(qbuf, qd_out.at[pl.ds(row * _CH, _CH)])
            return carry

        lax.fori_loop(0, n_ch, body, 0)

    return gather_k(node_attr, q, src2d, dst2d)


def _sc_scatter(rows1_in, rows2_in, dst2d, dsthi2d, z1, z2, e_pad):
    n = z1.shape[0]
    nhi = z2.shape[0]
    n_ch = e_pad // (_NW * _CH)
    stripe = n // _NS
    stripe2 = nhi // _NS
    mesh = plsc.VectorSubcoreMesh(core_axis_name="c", subcore_axis_name="s",
                                  num_cores=_NC, num_subcores=_NS)

    @functools.partial(
        pl.kernel,
        out_type=(jax.ShapeDtypeStruct((_NC, n, _D_OUT), jnp.float32),
                  jax.ShapeDtypeStruct((_NC, nhi, _D_OUT), jnp.float32)),
        mesh=mesh,
        scratch_types=(pltpu.VMEM((e_pad // (_NW * _CH), _CH), jnp.int32),
                       pltpu.VMEM((e_pad // (_NW * _CH), _CH), jnp.int32),
                       pltpu.VMEM((_CH, _D_OUT), jnp.float32),
                       pltpu.VMEM((_CH, _D_OUT), jnp.float32),
                       pltpu.VMEM_SHARED((n, _D_OUT), jnp.float32),
                       pltpu.VMEM_SHARED((nhi, _D_OUT), jnp.float32)),
    )
    def scatter_k(rows1_hbm, rows2_hbm, dst_hbm, dsthi_hbm, z1_hbm, z2_hbm,
                  p1_out, p2_out, dst_v, dsthi_v, rbuf1, rbuf2, acc1, acc2):
        cid = lax.axis_index("c")
        sid = lax.axis_index("s")
        wid = sid * _NC + cid
        ch0 = wid * n_ch
        pltpu.sync_copy(dst_hbm.at[pl.ds(ch0, n_ch)], dst_v)
        pltpu.sync_copy(dsthi_hbm.at[pl.ds(ch0, n_ch)], dsthi_v)
        pltpu.sync_copy(z1_hbm.at[pl.ds(sid * stripe, stripe)],
                        acc1.at[pl.ds(sid * stripe, stripe)])
        pltpu.sync_copy(z2_hbm.at[pl.ds(sid * stripe2, stripe2)],
                        acc2.at[pl.ds(sid * stripe2, stripe2)])
        plsc.subcore_barrier()

        def body(j, carry):
            row = ch0 + j
            pltpu.sync_copy(rows1_hbm.at[pl.ds(row * _CH, _CH)], rbuf1)
            pltpu.sync_copy(rows2_hbm.at[pl.ds(row * _CH, _CH)], rbuf2)
            pltpu.sync_copy(rbuf1, acc1.at[dst_v.at[j]], add=True)
            pltpu.sync_copy(rbuf2, acc2.at[dsthi_v.at[j]], add=True)
            return carry

        lax.fori_loop(0, n_ch, body, 0)
        plsc.subcore_barrier()
        pltpu.sync_copy(acc1.at[pl.ds(sid * stripe, stripe)],
                        p1_out.at[cid, pl.ds(sid * stripe, stripe)])
        pltpu.sync_copy(acc2.at[pl.ds(sid * stripe2, stripe2)],
                        p2_out.at[cid, pl.ds(sid * stripe2, stripe2)])

    return scatter_k(rows1_in, rows2_in, dst2d, dsthi2d, z1, z2)


def _tc_prep(node_attr, wq, wk, wdot):
    n = node_attr.shape[0]
    return pl.pallas_call(
        _prep_body,
        out_shape=(jax.ShapeDtypeStruct((n, _D_IN), jnp.float32),
                   jax.ShapeDtypeStruct((_D_IN * _D_SH, _D_Q), jnp.float32)),
    )(node_attr, wq, wk, wdot)


def _tc_dense(feat, qd, ea, sh, dstm, w1, b1, w2k, w2v, wkd, wv, e_real,
              e_pad):
    grid = e_pad // _BE
    full = lambda shape: pl.BlockSpec(shape, lambda i: (0, 0))
    return pl.pallas_call(
        functools.partial(_dense_body, e_real),
        grid=(grid,),
        in_specs=[
            pl.BlockSpec((_BE, _D_IN), lambda i: (i, 0)),
            pl.BlockSpec((_BE, _D_IN), lambda i: (i, 0)),
            pl.BlockSpec((_BE, 16), lambda i: (i, 0)),
            pl.BlockSpec((_BE, _D_SH), lambda i: (i, 0)),
            pl.BlockSpec((_BE, 1), lambda i: (i, 0)),
            full((16, 128)),
            full((1, 128)),
            full((64, 512)),
            full((64, 512)),
            full((512, _D_IN)),
            full((512, _D_OUT)),
        ],
        out_specs=(pl.BlockSpec((_BE, _D_OUT), lambda i: (i, 0)),
                   pl.BlockSpec((_BE, _D_OUT), lambda i: (i, 0))),
        out_shape=(jax.ShapeDtypeStruct((e_pad, _D_OUT), jnp.float32),
                   jax.ShapeDtypeStruct((e_pad, _D_OUT), jnp.float32)),
    )(feat, qd, ea, sh, dstm, w1, b1, w2k, w2v, wkd, wv)


def _tc_finalize(p1, p2):
    n = p1.shape[1]
    bn = 632 if n % 632 == 0 else n
    return pl.pallas_call(
        _fin_body,
        grid=(n // bn,),
        in_specs=[pl.BlockSpec((_NC, bn, _D_OUT), lambda i: (0, i, 0)),
                  pl.BlockSpec((_NC, bn, 1), lambda i: (0, i, 0))],
        out_specs=pl.BlockSpec((bn, _D_OUT), lambda i: (i, 0)),
        out_shape=jax.ShapeDtypeStruct((n, _D_OUT), jnp.float32),
    )(p1, p2)


def kernel(node_attr, edge_index, edge_attr, edge_sh,
           Wq, W1k, b1k, W2k, Wk, W1v, b1v, W2v, Wv, Wdot):
    n = node_attr.shape[0]
    e = edge_index.shape[1]
    align = _NW * _CH
    e_pad = ((e + align - 1) // align) * align
    if e_pad % _BE:
        e_pad = ((e_pad + _BE - 1) // _BE) * _BE
    pad = e_pad - e

    src2d = jnp.pad(edge_index[0], (0, pad)).reshape(-1, _CH)
    dst2d = jnp.pad(edge_index[1], (0, pad)).reshape(-1, _CH)
    ea = jnp.pad(edge_attr, ((0, pad), (0, 0)))
    sh = jnp.pad(edge_sh, ((0, pad), (0, 0)))

    wq_pad = jnp.pad(Wq, ((0, 0), (0, _D_IN - _D_Q)))
    w1 = jnp.concatenate([W1k, W1v], axis=1)
    b1 = jnp.concatenate([b1k, b1v])[None, :]
    w2k = W2k.reshape(64, _D_IN, _D_SH).transpose(0, 2, 1).reshape(64, 512)
    w2v = W2v.reshape(64, _D_IN, _D_SH).transpose(0, 2, 1).reshape(64, 512)
    wvp = Wv.reshape(_D_IN, _D_SH, _D_OUT).transpose(1, 0, 2).reshape(512, _D_OUT)

    n_pad = ((n + _NS * 8 - 1) // (_NS * 8)) * (_NS * 8)

    q, wkd = _tc_prep(node_attr, wq_pad, Wk, Wdot)
    wkdp = wkd.reshape(_D_IN, _D_SH, _D_Q).transpose(1, 0, 2).reshape(512, _D_Q)
    wkdp = jnp.pad(wkdp, ((0, 0), (0, _D_IN - _D_Q)))

    feat, qd = _sc_gather(node_attr, q, src2d, dst2d, e_pad)
    dst_flat = dst2d.reshape(-1)
    dstm = (dst_flat % _D_OUT).astype(jnp.int32)[:, None]
    dsthi2d = (dst_flat // _D_OUT).astype(jnp.int32).reshape(-1, _CH)
    rows1, rows2 = _tc_dense(feat, qd, ea, sh, dstm, w1, b1, w2k, w2v,
                             wkdp, wvp, e, e_pad)
    nhi = ((n_pad // _D_OUT + _NS * 8 - 1) // (_NS * 8)) * (_NS * 8)
    z1 = jnp.zeros((n_pad, _D_OUT), jnp.float32)
    z2 = jnp.zeros((nhi, _D_OUT), jnp.float32)
    p1, p2 = _sc_scatter(rows1, rows2, dst2d, dsthi2d, z1, z2, e_pad)
    den = p2.reshape(_NC, -1, 1)[:, :n_pad]
    return _tc_finalize(p1, den)[:n]


# speedup vs baseline: 4.2822x; 1.2719x over previous
import functools

import jax
import jax.numpy as jnp
from jax import lax
from jax.experimental import pallas as pl
from jax.experimental.pallas import tpu as pltpu
from jax.experimental.pallas import tpu_sc as plsc

_NC = 2
_NS = 16
_NW = _NC * _NS
_CH = 128
_BE = 1024
_D_IN = 128
_D_SH = 4
_D_Q = 80
_D_OUT = 128


def _prep_body(na_ref, wq_ref, wk_ref, wdot_ref, q_ref, wkd_ref):
    q_ref[...] = jnp.dot(na_ref[...], wq_ref[...],
                         preferred_element_type=jnp.float32)
    wkd_ref[...] = lax.dot_general(
        wk_ref[...], wdot_ref[...], (((1,), (1,)), ((), ())),
        preferred_element_type=jnp.float32)


def _dense_body(e_real, feat_ref, qd_ref, ea_ref, sh_ref, dstm_ref, w1_ref,
                b1_ref, w2k_ref, w2v_ref, wkd_ref, wv_ref, o1_ref, o2_ref):
    b = feat_ref.shape[0]
    h = jnp.maximum(
        jnp.dot(ea_ref[...], w1_ref[...], preferred_element_type=jnp.float32)
        + b1_ref[...], 0.0)
    wk = jnp.dot(h[:, :64], w2k_ref[...], preferred_element_type=jnp.float32)
    wv = jnp.dot(h[:, 64:], w2v_ref[...], preferred_element_type=jnp.float32)
    f = feat_ref[...]
    sh = sh_ref[...]
    tp = jnp.concatenate([f * sh[:, s:s + 1] for s in range(_D_SH)], axis=1)
    kd = jnp.dot(tp * wk, wkd_ref[...], preferred_element_type=jnp.float32)
    v = jnp.dot(tp * wv, wv_ref[...], preferred_element_type=jnp.float32)
    alpha = jnp.sum(qd_ref[...] * kd, axis=1, keepdims=True) * (_D_Q ** -0.5)
    rows = lax.broadcasted_iota(jnp.int32, (b, 1), 0) + pl.program_id(0) * b
    ealpha = jnp.where(rows < e_real, jnp.exp(alpha), 0.0)
    o1_ref[...] = ealpha * v
    lanes = lax.broadcasted_iota(jnp.int32, (b, _D_OUT), 1)
    o2_ref[...] = jnp.where(lanes == dstm_ref[...], ealpha, 0.0)


def _fin_body(p1_ref, p2_ref, o_ref):
    num = p1_ref[0] + p1_ref[1]
    den = p2_ref[0] + p2_ref[1]
    safe = jnp.where(den > 0.0, den, 1.0)
    o_ref[...] = jnp.where(den > 0.0, num / safe, 0.0)


def _sc_gather(node_attr, q, src2d, dst2d, e_pad):
    n_ch = e_pad // (_NW * _CH)
    mesh = plsc.VectorSubcoreMesh(core_axis_name="c", subcore_axis_name="s",
                                  num_cores=_NC, num_subcores=_NS)

    @functools.partial(
        pl.kernel,
        out_type=(jax.ShapeDtypeStruct((e_pad, _D_IN), jnp.float32),
                  jax.ShapeDtypeStruct((e_pad, _D_IN), jnp.float32)),
        mesh=mesh,
        scratch_types=(pltpu.VMEM((e_pad // (_NW * _CH), _CH), jnp.int32),
                       pltpu.VMEM((e_pad // (_NW * _CH), _CH), jnp.int32),
                       pltpu.VMEM((2, _CH, _D_IN), jnp.float32),
                       pltpu.VMEM((2, _CH, _D_IN), jnp.float32),
                       pltpu.SemaphoreType.DMA((2,)),
                       pltpu.SemaphoreType.DMA((2,)),
                       pltpu.SemaphoreType.DMA((2,)),
                       pltpu.SemaphoreType.DMA((2,))),
    )
    def gather_k(node_hbm, q_hbm, src_hbm, dst_hbm, feat_out, qd_out,
                 src_v, dst_v, fbuf, qbuf, sfg, sfw, sqg, sqw):
        wid = lax.axis_index("s") * _NC + lax.axis_index("c")
        ch0 = wid * n_ch
        pltpu.sync_copy(src_hbm.at[pl.ds(ch0, n_ch)], src_v)
        pltpu.sync_copy(dst_hbm.at[pl.ds(ch0, n_ch)], dst_v)

        def g_wait(b):
            pltpu.make_async_copy(node_hbm.at[pl.ds(0, _CH)], fbuf.at[b],
                                  sfg.at[b]).wait()
            pltpu.make_async_copy(q_hbm.at[pl.ds(0, _CH)], qbuf.at[b],
                                  sqg.at[b]).wait()

        def w_wait(b):
            pltpu.make_async_copy(fbuf.at[b], feat_out.at[pl.ds(0, _CH)],
                                  sfw.at[b]).wait()
            pltpu.make_async_copy(qbuf.at[b], qd_out.at[pl.ds(0, _CH)],
                                  sqw.at[b]).wait()

        def g_start(b, j):
            pltpu.async_copy(node_hbm.at[src_v.at[j]], fbuf.at[b], sfg.at[b])
            pltpu.async_copy(q_hbm.at[dst_v.at[j]], qbuf.at[b], sqg.at[b])

        def w_start(b, j):
            row = ch0 + j
            pltpu.async_copy(fbuf.at[b], feat_out.at[pl.ds(row * _CH, _CH)],
                             sfw.at[b])
            pltpu.async_copy(qbuf.at[b], qd_out.at[pl.ds(row * _CH, _CH)],
                             sqw.at[b])

        for b in range(2):
            g_start(b, b)

        def body(j2, carry):
            j = j2 * 2
            for b in range(2):
                g_wait(b)
                w_start(b, j + b)
            for b in range(2):
                w_wait(b)
                g_start(b, j + b + 2)
            return carry

        lax.fori_loop(0, (n_ch - 2) // 2, body, 0)
        jlast = n_ch - 2
        for b in range(2):
            g_wait(b)
            w_start(b, jlast + b)
        for b in range(2):
            w_wait(b)

    return gather_k(node_attr, q, src2d, dst2d)


def _sc_scatter(rows1_in, rows2_in, dst2d, dsthi2d, z1, z2, e_pad):
    n = z1.shape[0]
    nhi = z2.shape[0]
    n_ch = e_pad // (_NW * _CH)
    stripe = n // _NS
    stripe2 = nhi // _NS
    mesh = plsc.VectorSubcoreMesh(core_axis_name="c", subcore_axis_name="s",
                                  num_cores=_NC, num_subcores=_NS)

    @functools.partial(
        pl.kernel,
        out_type=(jax.ShapeDtypeStruct((_NC, n, _D_OUT), jnp.float32),
                  jax.ShapeDtypeStruct((_NC, nhi, _D_OUT), jnp.float32)),
        mesh=mesh,
        scratch_types=(pltpu.VMEM((e_pad // (_NW * _CH), _CH), jnp.int32),
                       pltpu.VMEM((e_pad // (_NW * _CH), _CH), jnp.int32),
                       pltpu.VMEM((_CH, _D_OUT), jnp.float32),
                       pltpu.VMEM((_CH, _D_OUT), jnp.float32),
                       pltpu.VMEM_SHARED((n, _D_OUT), jnp.float32),
                       pltpu.VMEM_SHARED((nhi, _D_OUT), jnp.float32)),
    )
    def scatter_k(rows1_hbm, rows2_hbm, dst_hbm, dsthi_hbm, z1_hbm, z2_hbm,
                  p1_out, p2_out, dst_v, dsthi_v, rbuf1, rbuf2, acc1, acc2):
        cid = lax.axis_index("c")
        sid = lax.axis_index("s")
        wid = sid * _NC + cid
        ch0 = wid * n_ch
        pltpu.sync_copy(dst_hbm.at[pl.ds(ch0, n_ch)], dst_v)
        pltpu.sync_copy(dsthi_hbm.at[pl.ds(ch0, n_ch)], dsthi_v)
        pltpu.sync_copy(z1_hbm.at[pl.ds(sid * stripe, stripe)],
                        acc1.at[pl.ds(sid * stripe, stripe)])
        pltpu.sync_copy(z2_hbm.at[pl.ds(sid * stripe2, stripe2)],
                        acc2.at[pl.ds(sid * stripe2, stripe2)])
        plsc.subcore_barrier()

        def body(j, carry):
            row = ch0 + j
            pltpu.sync_copy(rows1_hbm.at[pl.ds(row * _CH, _CH)], rbuf1)
            pltpu.sync_copy(rows2_hbm.at[pl.ds(row * _CH, _CH)], rbuf2)
            pltpu.sync_copy(rbuf1, acc1.at[dst_v.at[j]], add=True)
            pltpu.sync_copy(rbuf2, acc2.at[dsthi_v.at[j]], add=True)
            return carry

        lax.fori_loop(0, n_ch, body, 0)
        plsc.subcore_barrier()
        pltpu.sync_copy(acc1.at[pl.ds(sid * stripe, stripe)],
                        p1_out.at[cid, pl.ds(sid * stripe, stripe)])
        pltpu.sync_copy(acc2.at[pl.ds(sid * stripe2, stripe2)],
                        p2_out.at[cid, pl.ds(sid * stripe2, stripe2)])

    return scatter_k(rows1_in, rows2_in, dst2d, dsthi2d, z1, z2)


def _tc_prep(node_attr, wq, wk, wdot):
    n = node_attr.shape[0]
    return pl.pallas_call(
        _prep_body,
        out_shape=(jax.ShapeDtypeStruct((n, _D_IN), jnp.float32),
                   jax.ShapeDtypeStruct((_D_IN * _D_SH, _D_Q), jnp.float32)),
    )(node_attr, wq, wk, wdot)


def _tc_dense(feat, qd, ea, sh, dstm, w1, b1, w2k, w2v, wkd, wv, e_real,
              e_pad):
    grid = e_pad // _BE
    full = lambda shape: pl.BlockSpec(shape, lambda i: (0, 0))
    return pl.pallas_call(
        functools.partial(_dense_body, e_real),
        grid=(grid,),
        in_specs=[
            pl.BlockSpec((_BE, _D_IN), lambda i: (i, 0)),
            pl.BlockSpec((_BE, _D_IN), lambda i: (i, 0)),
            pl.BlockSpec((_BE, 16), lambda i: (i, 0)),
            pl.BlockSpec((_BE, _D_SH), lambda i: (i, 0)),
            pl.BlockSpec((_BE, 1), lambda i: (i, 0)),
            full((16, 128)),
            full((1, 128)),
            full((64, 512)),
            full((64, 512)),
            full((512, _D_IN)),
            full((512, _D_OUT)),
        ],
        out_specs=(pl.BlockSpec((_BE, _D_OUT), lambda i: (i, 0)),
                   pl.BlockSpec((_BE, _D_OUT), lambda i: (i, 0))),
        out_shape=(jax.ShapeDtypeStruct((e_pad, _D_OUT), jnp.float32),
                   jax.ShapeDtypeStruct((e_pad, _D_OUT), jnp.float32)),
    )(feat, qd, ea, sh, dstm, w1, b1, w2k, w2v, wkd, wv)


def _tc_finalize(p1, p2):
    n = p1.shape[1]
    bn = 632 if n % 632 == 0 else n
    return pl.pallas_call(
        _fin_body,
        grid=(n // bn,),
        in_specs=[pl.BlockSpec((_NC, bn, _D_OUT), lambda i: (0, i, 0)),
                  pl.BlockSpec((_NC, bn, 1), lambda i: (0, i, 0))],
        out_specs=pl.BlockSpec((bn, _D_OUT), lambda i: (i, 0)),
        out_shape=jax.ShapeDtypeStruct((n, _D_OUT), jnp.float32),
    )(p1, p2)


def kernel(node_attr, edge_index, edge_attr, edge_sh,
           Wq, W1k, b1k, W2k, Wk, W1v, b1v, W2v, Wv, Wdot):
    n = node_attr.shape[0]
    e = edge_index.shape[1]
    align = _NW * _CH
    e_pad = ((e + align - 1) // align) * align
    if e_pad % _BE:
        e_pad = ((e_pad + _BE - 1) // _BE) * _BE
    pad = e_pad - e

    src2d = jnp.pad(edge_index[0], (0, pad)).reshape(-1, _CH)
    dst2d = jnp.pad(edge_index[1], (0, pad)).reshape(-1, _CH)
    ea = jnp.pad(edge_attr, ((0, pad), (0, 0)))
    sh = jnp.pad(edge_sh, ((0, pad), (0, 0)))

    wq_pad = jnp.pad(Wq, ((0, 0), (0, _D_IN - _D_Q)))
    w1 = jnp.concatenate([W1k, W1v], axis=1)
    b1 = jnp.concatenate([b1k, b1v])[None, :]
    w2k = W2k.reshape(64, _D_IN, _D_SH).transpose(0, 2, 1).reshape(64, 512)
    w2v = W2v.reshape(64, _D_IN, _D_SH).transpose(0, 2, 1).reshape(64, 512)
    wvp = Wv.reshape(_D_IN, _D_SH, _D_OUT).transpose(1, 0, 2).reshape(512, _D_OUT)

    n_pad = ((n + _NS * 8 - 1) // (_NS * 8)) * (_NS * 8)

    q, wkd = _tc_prep(node_attr, wq_pad, Wk, Wdot)
    wkdp = wkd.reshape(_D_IN, _D_SH, _D_Q).transpose(1, 0, 2).reshape(512, _D_Q)
    wkdp = jnp.pad(wkdp, ((0, 0), (0, _D_IN - _D_Q)))

    feat, qd = _sc_gather(node_attr, q, src2d, dst2d, e_pad)
    dst_flat = dst2d.reshape(-1)
    dstm = (dst_flat % _D_OUT).astype(jnp.int32)[:, None]
    dsthi2d = (dst_flat // _D_OUT).astype(jnp.int32).reshape(-1, _CH)
    rows1, rows2 = _tc_dense(feat, qd, ea, sh, dstm, w1, b1, w2k, w2v,
                             wkdp, wvp, e, e_pad)
    nhi = ((n_pad // _D_OUT + _NS * 8 - 1) // (_NS * 8)) * (_NS * 8)
    z1 = jnp.zeros((n_pad, _D_OUT), jnp.float32)
    z2 = jnp.zeros((nhi, _D_OUT), jnp.float32)
    p1, p2 = _sc_scatter(rows1, rows2, dst2d, dsthi2d, z1, z2, e_pad)
    den = p2.reshape(_NC, -1, 1)[:, :n_pad]
    return _tc_finalize(p1, den)[:n]
